# baseline (speedup 1.0000x reference)
import functools
import math

import jax
import jax.numpy as jnp
from jax import lax
from jax.experimental import pallas as pl
from jax.experimental.pallas import tpu as pltpu
from jax.experimental.pallas import tpu_sc as plsc

_N = 10000
_E = 320000
_HID = 128
_HH = _HID // 2
_NE = 8
_NCORE = 2
_NSUB = 16
_NW = _NCORE * _NSUB
_K = 128
_CH = 80
_EPW = _K * _CH
_EPAD = _NW * _EPW - _E
_NP = 10240
_RPT = _NP // _NSUB
_BLK = 1000
_COEF = 0.01
_NUM_LAYERS = 4


def _gate_body(x_ref, noise_ref, win_ref, bin_ref, wg_ref, wn_ref,
               h_ref, raw_ref, gates_ref, implo_ref):
    xb = x_ref[...]
    h = jnp.maximum(
        jnp.dot(xb, win_ref[...], preferred_element_type=jnp.float32)
        + bin_ref[...], 0.0)
    h_ref[0] = h[:, :_HH]
    h_ref[1] = h[:, _HH:]
    clean = jnp.dot(h, wg_ref[...], preferred_element_type=jnp.float32)
    sp = jnp.dot(h, wn_ref[...], preferred_element_type=jnp.float32)
    nstd = jnp.maximum(sp, 0.0) + jnp.log1p(jnp.exp(-jnp.abs(sp))) + 1e-2
    raw = clean + noise_ref[...] * nstd
    raw_ref[...] = raw

    ids = lax.broadcasted_iota(jnp.int32, raw.shape, 1)
    v1 = jnp.max(raw, axis=1, keepdims=True)
    i1 = jnp.min(jnp.where(raw >= v1, ids, _NE), axis=1, keepdims=True)
    masked = jnp.where(ids == i1, -jnp.inf, raw)
    v2 = jnp.max(masked, axis=1, keepdims=True)
    i2 = jnp.min(jnp.where(masked >= v2, ids, _NE), axis=1, keepdims=True)
    e = jnp.exp(v2 - v1)
    g1 = 1.0 / (1.0 + e)
    g2 = e / (1.0 + e)
    gates = jnp.where(ids == i1, g1, 0.0) + jnp.where(ids == i2, g2, 0.0)
    gates_ref[...] = gates

    @pl.when(pl.program_id(0) == 0)
    def _():
        implo_ref[...] = jnp.zeros_like(implo_ref)

    imp = jnp.sum(gates, axis=0)[None, :]
    load = jnp.sum((gates > 0.0).astype(jnp.float32), axis=0)[None, :]
    implo_ref[...] += jnp.concatenate([imp, load], axis=0)


def _gating_call(x, noise, w_in, b_in, w_gate, w_noise):
    grid = (_N // _BLK,)
    return pl.pallas_call(
        _gate_body,
        grid=grid,
        in_specs=[
            pl.BlockSpec((_BLK, _HID), lambda i: (i, 0)),
            pl.BlockSpec((_BLK, _NE), lambda i: (i, 0)),
            pl.BlockSpec((_HID, _HID), lambda i: (0, 0)),
            pl.BlockSpec((1, _HID), lambda i: (0, 0)),
            pl.BlockSpec((_HID, _NE), lambda i: (0, 0)),
            pl.BlockSpec((_HID, _NE), lambda i: (0, 0)),
        ],
        out_specs=[
            pl.BlockSpec((2, _BLK, _HH), lambda i: (0, i, 0)),
            pl.BlockSpec((_BLK, _NE), lambda i: (i, 0)),
            pl.BlockSpec((_BLK, _NE), lambda i: (i, 0)),
            pl.BlockSpec((2, _NE), lambda i: (0, 0)),
        ],
        out_shape=[
            jax.ShapeDtypeStruct((2, _N, _HH), jnp.float32),
            jax.ShapeDtypeStruct((_N, _NE), jnp.float32),
            jax.ShapeDtypeStruct((_N, _NE), jnp.float32),
            jax.ShapeDtypeStruct((2, _NE), jnp.float32),
        ],
    )(x, noise, w_in, b_in, w_gate, w_noise)


def _sc_aggregate(h2, ei, z64, z16, ones16):
    mesh = plsc.VectorSubcoreMesh(core_axis_name="c", subcore_axis_name="s")

    @functools.partial(
        pl.kernel,
        out_type=[
            jax.ShapeDtypeStruct((2, _NCORE, _NP, _HH), jnp.float32),
            jax.ShapeDtypeStruct((_NCORE, _NP, 16), jnp.float32),
        ],
        mesh=mesh,
        compiler_params=pltpu.CompilerParams(use_tc_tiling_on_sc=False),
        scratch_types=[
            pltpu.VMEM((_CH, _K), jnp.int32),
            pltpu.VMEM((_CH, _K), jnp.int32),
            [pltpu.VMEM((_K, _HH), jnp.float32) for _ in range(4)],
            pltpu.VMEM((_K, 16), jnp.float32),
            pltpu.VMEM_SHARED((_NP, _HH), jnp.float32),
            pltpu.VMEM_SHARED((_NP, 16), jnp.float32),
            [pltpu.SemaphoreType.DMA for _ in range(4)],
            [pltpu.SemaphoreType.DMA for _ in range(4)],
        ],
    )
    def body(h_hbm, ei_hbm, z64_hbm, z16_hbm, ones_hbm,
             aggp_hbm, degp_hbm,
             sidx, didx, rows, ones_v, agg_sh, deg_sh, gsem, ssem):
        cid = lax.axis_index("c")
        sid = lax.axis_index("s")
        w = cid * _NSUB + sid
        rbase = sid * _RPT

        pre = [pltpu.async_copy(ones_hbm, ones_v, gsem[0]),
               pltpu.async_copy(ei_hbm.at[0, w], sidx, gsem[1]),
               pltpu.async_copy(ei_hbm.at[1, w], didx, gsem[2])]

        for half in range(2):
            first = half == 0
            htab = h_hbm.at[half]

            pltpu.sync_copy(z64_hbm, agg_sh.at[pl.ds(rbase, _RPT)])
            if first:
                pltpu.sync_copy(z16_hbm, deg_sh.at[pl.ds(rbase, _RPT)])
                for p in pre:
                    p.wait()
            plsc.subcore_barrier()

            def start_gather(c, rows, sem):
                pltpu.async_copy(htab.at[sidx.at[c]], rows, sem)

            def wait_gather(c, rows, sem):
                pltpu.make_async_copy(htab.at[sidx.at[c]], rows, sem).wait()

            def start_scatter(c, rows, sem):
                pltpu.async_copy(rows, agg_sh.at[didx.at[c]], sem, add=True)
                if first:
                    pltpu.async_copy(ones_v, deg_sh.at[didx.at[c]], sem,
                                     add=True)

            def wait_scatter(c, rows, sem):
                pltpu.make_async_copy(rows, agg_sh.at[didx.at[c]], sem).wait()
                if first:
                    pltpu.make_async_copy(ones_v, deg_sh.at[didx.at[c]],
                                          sem).wait()

            for j in range(4):
                start_gather(j, rows[j], gsem[j])

            def step(i, carry):
                c = 4 * i
                for j in range(4):
                    wait_gather(c + j, rows[j], gsem[j])
                    start_scatter(c + j, rows[j], ssem[j])
                for j in range(4):
                    wait_scatter(c + j, rows[j], ssem[j])
                    start_gather(c + j + 4, rows[j], gsem[j])
                return carry

            lax.fori_loop(0, _CH // 4 - 1, step, 0)

            c = _CH - 4
            for j in range(4):
                wait_gather(c + j, rows[j], gsem[j])
                start_scatter(c + j, rows[j], ssem[j])
            for j in range(4):
                wait_scatter(c + j, rows[j], ssem[j])

            plsc.subcore_barrier()
            pltpu.sync_copy(agg_sh.at[pl.ds(rbase, _RPT)],
                            aggp_hbm.at[half, cid, pl.ds(rbase, _RPT)])
            if first:
                pltpu.sync_copy(deg_sh.at[pl.ds(rbase, _RPT)],
                                degp_hbm.at[cid, pl.ds(rbase, _RPT)])

    return body(h2, ei, z64, z16, ones16)


def _moe_body(p_ref, degp_ref, gates_ref, implo_ref, we_ref, be_ref,
              wo_ref, bo_ref, out_ref, lb_ref):
    p = p_ref[...]
    agg_sum = jnp.concatenate([p[0, 0] + p[0, 1], p[1, 0] + p[1, 1]], axis=1)
    deg = degp_ref[0, :, 0] + degp_ref[1, :, 0]
    agg = agg_sum / jnp.maximum(deg, 1.0)[:, None]
    gates = gates_ref[...]
    y = jnp.zeros_like(agg)
    for e in range(_NE):
        eo = jnp.maximum(
            jnp.dot(agg, we_ref[e], preferred_element_type=jnp.float32)
            + be_ref[e][None, :], 0.0)
        y = y + gates[:, e][:, None] * eo
    out_ref[...] = (jnp.dot(y, wo_ref[...], preferred_element_type=jnp.float32)
                    + bo_ref[...])

    @pl.when(pl.program_id(0) == 0)
    def _():
        implo = implo_ref[...]

        def cv2(v):
            m = jnp.mean(v)
            var = jnp.sum((v - m) ** 2) / (v.shape[0] - 1)
            return var / (m * m + 1e-10)

        scale = _COEF / math.ceil((_NUM_LAYERS - 2) / 2)
        lb_ref[...] = (scale * (cv2(implo[0]) + cv2(implo[1]))).reshape(1, 1)


def _moe_call(aggp, degp, gates, implo, w_expert, b_expert, w_out, b_out):
    grid = (_N // _BLK,)
    d_out = w_out.shape[1]
    return pl.pallas_call(
        _moe_body,
        grid=grid,
        in_specs=[
            pl.BlockSpec((2, _NCORE, _BLK, _HH), lambda i: (0, 0, i, 0)),
            pl.BlockSpec((_NCORE, _BLK, 16), lambda i: (0, i, 0)),
            pl.BlockSpec((_BLK, _NE), lambda i: (i, 0)),
            pl.BlockSpec((2, _NE), lambda i: (0, 0)),
            pl.BlockSpec((_NE, _HID, _HID), lambda i: (0, 0, 0)),
            pl.BlockSpec((_NE, _HID), lambda i: (0, 0)),
            pl.BlockSpec((_HID, d_out), lambda i: (0, 0)),
            pl.BlockSpec((1, d_out), lambda i: (0, 0)),
        ],
        out_specs=[
            pl.BlockSpec((_BLK, d_out), lambda i: (i, 0)),
            pl.BlockSpec((1, 1), lambda i: (0, 0)),
        ],
        out_shape=[
            jax.ShapeDtypeStruct((_N, d_out), jnp.float32),
            jax.ShapeDtypeStruct((1, 1), jnp.float32),
        ],
    )(aggp, degp, gates, implo, w_expert, b_expert, w_out, b_out)


def kernel(x, edge_index, noise, W_in, b_in, w_gate, w_noise,
           W_expert, b_expert, W_out, b_out):
    h2, raw_logits, gates, implo = _gating_call(
        x, noise, W_in, b_in.reshape(1, -1), w_gate, w_noise)
    pad = jnp.concatenate(
        [jnp.zeros((1, _EPAD), jnp.int32),
         _N + (jnp.arange(_EPAD, dtype=jnp.int32) % (_NP - _N))[None, :]],
        axis=0)
    ei = jnp.concatenate([edge_index, pad], axis=1).reshape(2, _NW, _CH, _K)
    z64 = jnp.zeros((_RPT, _HH), jnp.float32)
    z16 = jnp.zeros((_RPT, 16), jnp.float32)
    ones16 = jnp.ones((_K, 16), jnp.float32)
    aggp, degp = _sc_aggregate(h2, ei, z64, z16, ones16)
    out, lb = _moe_call(aggp, degp, gates, implo,
                        W_expert, b_expert, W_out, b_out.reshape(1, -1))
    return out, lb.reshape(()), raw_logits


# speedup vs baseline: 1.8483x; 1.8483x over previous
import functools
import math

import jax
import jax.numpy as jnp
from jax import lax
from jax.experimental import pallas as pl
from jax.experimental.pallas import tpu as pltpu
from jax.experimental.pallas import tpu_sc as plsc

_N = 10000
_E = 320000
_HID = 128
_HH = _HID // 2
_NE = 8
_NCORE = 2
_NSUB = 16
_NW = _NCORE * _NSUB
_EPW = _E // _NW
_K = 80
_CH = _EPW // _K
_NP = 10240
_RPT = _NP // _NSUB
_BLK = 1000
_COEF = 0.01
_NUM_LAYERS = 4


def _gate_body(x_ref, noise_ref, win_ref, bin_ref, wg_ref, wn_ref,
               h_ref, raw_ref, gates_ref, implo_ref):
    xb = x_ref[...]
    h = jnp.maximum(
        jnp.dot(xb, win_ref[...], preferred_element_type=jnp.float32)
        + bin_ref[...], 0.0)
    h_ref[0] = h[:, :_HH]
    h_ref[1] = h[:, _HH:]
    clean = jnp.dot(h, wg_ref[...], preferred_element_type=jnp.float32)
    sp = jnp.dot(h, wn_ref[...], preferred_element_type=jnp.float32)
    nstd = jnp.maximum(sp, 0.0) + jnp.log1p(jnp.exp(-jnp.abs(sp))) + 1e-2
    raw = clean + noise_ref[...] * nstd
    raw_ref[...] = raw

    ids = lax.broadcasted_iota(jnp.int32, raw.shape, 1)
    v1 = jnp.max(raw, axis=1, keepdims=True)
    i1 = jnp.min(jnp.where(raw >= v1, ids, _NE), axis=1, keepdims=True)
    masked = jnp.where(ids == i1, -jnp.inf, raw)
    v2 = jnp.max(masked, axis=1, keepdims=True)
    i2 = jnp.min(jnp.where(masked >= v2, ids, _NE), axis=1, keepdims=True)
    e = jnp.exp(v2 - v1)
    g1 = 1.0 / (1.0 + e)
    g2 = e / (1.0 + e)
    gates = jnp.where(ids == i1, g1, 0.0) + jnp.where(ids == i2, g2, 0.0)
    gates_ref[...] = gates

    @pl.when(pl.program_id(0) == 0)
    def _():
        implo_ref[...] = jnp.zeros_like(implo_ref)

    imp = jnp.sum(gates, axis=0)[None, :]
    load = jnp.sum((gates > 0.0).astype(jnp.float32), axis=0)[None, :]
    implo_ref[...] += jnp.concatenate([imp, load], axis=0)


def _gating_call(x, noise, w_in, b_in, w_gate, w_noise):
    grid = (_N // _BLK,)
    return pl.pallas_call(
        _gate_body,
        grid=grid,
        in_specs=[
            pl.BlockSpec((_BLK, _HID), lambda i: (i, 0)),
            pl.BlockSpec((_BLK, _NE), lambda i: (i, 0)),
            pl.BlockSpec((_HID, _HID), lambda i: (0, 0)),
            pl.BlockSpec((1, _HID), lambda i: (0, 0)),
            pl.BlockSpec((_HID, _NE), lambda i: (0, 0)),
            pl.BlockSpec((_HID, _NE), lambda i: (0, 0)),
        ],
        out_specs=[
            pl.BlockSpec((2, _BLK, _HH), lambda i: (0, i, 0)),
            pl.BlockSpec((_BLK, _NE), lambda i: (i, 0)),
            pl.BlockSpec((_BLK, _NE), lambda i: (i, 0)),
            pl.BlockSpec((2, _NE), lambda i: (0, 0)),
        ],
        out_shape=[
            jax.ShapeDtypeStruct((2, _N, _HH), jnp.float32),
            jax.ShapeDtypeStruct((_N, _NE), jnp.float32),
            jax.ShapeDtypeStruct((_N, _NE), jnp.float32),
            jax.ShapeDtypeStruct((2, _NE), jnp.float32),
        ],
    )(x, noise, w_in, b_in, w_gate, w_noise)


def _sc_aggregate(h2, ei, z64, z16, ones16):
    mesh = plsc.VectorSubcoreMesh(core_axis_name="c", subcore_axis_name="s")

    @functools.partial(
        pl.kernel,
        out_type=[
            jax.ShapeDtypeStruct((2, _NCORE, _NP, _HH), jnp.float32),
            jax.ShapeDtypeStruct((_NCORE, _NP, 16), jnp.float32),
        ],
        mesh=mesh,
        compiler_params=pltpu.CompilerParams(use_tc_tiling_on_sc=False),
        scratch_types=[
            pltpu.VMEM((_CH, _K), jnp.int32),
            pltpu.VMEM((_CH, _K), jnp.int32),
            [pltpu.VMEM((_K, _HH), jnp.float32) for _ in range(2)],
            pltpu.VMEM((_K, 16), jnp.float32),
            pltpu.VMEM_SHARED((_NP, _HH), jnp.float32),
            pltpu.VMEM_SHARED((_NP, 16), jnp.float32),
            [pltpu.SemaphoreType.DMA for _ in range(2)],
            [pltpu.SemaphoreType.DMA for _ in range(2)],
        ],
    )
    def body(h_hbm, ei_hbm, z64_hbm, z16_hbm, ones_hbm,
             aggp_hbm, degp_hbm,
             sidx, didx, rows, ones_v, agg_sh, deg_sh, gsem, ssem):
        cid = lax.axis_index("c")
        sid = lax.axis_index("s")
        w = cid * _NSUB + sid
        rbase = sid * _RPT

        pltpu.sync_copy(ones_hbm, ones_v)
        pltpu.sync_copy(ei_hbm.at[0, w], sidx)
        pltpu.sync_copy(ei_hbm.at[1, w], didx)

        for half in range(2):
            first = half == 0
            htab = h_hbm.at[half]

            pltpu.sync_copy(z64_hbm, agg_sh.at[pl.ds(rbase, _RPT)])
            if first:
                pltpu.sync_copy(z16_hbm, deg_sh.at[pl.ds(rbase, _RPT)])
            plsc.subcore_barrier()

            def start_gather(c, rows, sem):
                pltpu.async_copy(htab.at[sidx.at[c]], rows, sem)

            def wait_gather(c, rows, sem):
                pltpu.make_async_copy(htab.at[sidx.at[c]], rows, sem).wait()

            def start_scatter(c, rows, sem):
                pltpu.async_copy(rows, agg_sh.at[didx.at[c]], sem, add=True)
                if first:
                    pltpu.async_copy(ones_v, deg_sh.at[didx.at[c]], sem,
                                     add=True)

            def wait_scatter(c, rows, sem):
                pltpu.make_async_copy(rows, agg_sh.at[didx.at[c]], sem).wait()
                if first:
                    pltpu.make_async_copy(ones_v, deg_sh.at[didx.at[c]],
                                          sem).wait()

            start_gather(0, rows[0], gsem[0])

            def step(i, carry):
                c0 = 2 * i
                c1 = c0 + 1
                c2 = c0 + 2
                wait_gather(c0, rows[0], gsem[0])
                start_scatter(c0, rows[0], ssem[0])
                start_gather(c1, rows[1], gsem[1])
                wait_gather(c1, rows[1], gsem[1])
                start_scatter(c1, rows[1], ssem[1])
                wait_scatter(c0, rows[0], ssem[0])
                start_gather(c2, rows[0], gsem[0])
                wait_scatter(c1, rows[1], ssem[1])
                return carry

            lax.fori_loop(0, (_CH - 1) // 2, step, 0)

            last = _CH - 1
            wait_gather(last, rows[0], gsem[0])
            start_scatter(last, rows[0], ssem[0])
            wait_scatter(last, rows[0], ssem[0])

            plsc.subcore_barrier()
            pltpu.sync_copy(agg_sh.at[pl.ds(rbase, _RPT)],
                            aggp_hbm.at[half, cid, pl.ds(rbase, _RPT)])
            if first:
                pltpu.sync_copy(deg_sh.at[pl.ds(rbase, _RPT)],
                                degp_hbm.at[cid, pl.ds(rbase, _RPT)])

    return body(h2, ei, z64, z16, ones16)


def _moe_body(p_ref, degp_ref, gates_ref, implo_ref, we_ref, be_ref,
              wo_ref, bo_ref, out_ref, lb_ref):
    p = p_ref[...]
    agg_sum = jnp.concatenate([p[0, 0] + p[0, 1], p[1, 0] + p[1, 1]], axis=1)
    deg = degp_ref[0, :, 0] + degp_ref[1, :, 0]
    agg = agg_sum / jnp.maximum(deg, 1.0)[:, None]
    gates = gates_ref[...]
    y = jnp.zeros_like(agg)
    for e in range(_NE):
        eo = jnp.maximum(
            jnp.dot(agg, we_ref[e], preferred_element_type=jnp.float32)
            + be_ref[e][None, :], 0.0)
        y = y + gates[:, e][:, None] * eo
    out_ref[...] = (jnp.dot(y, wo_ref[...], preferred_element_type=jnp.float32)
                    + bo_ref[...])

    @pl.when(pl.program_id(0) == 0)
    def _():
        implo = implo_ref[...]

        def cv2(v):
            m = jnp.mean(v)
            var = jnp.sum((v - m) ** 2) / (v.shape[0] - 1)
            return var / (m * m + 1e-10)

        scale = _COEF / math.ceil((_NUM_LAYERS - 2) / 2)
        lb_ref[...] = (scale * (cv2(implo[0]) + cv2(implo[1]))).reshape(1, 1)


def _moe_call(aggp, degp, gates, implo, w_expert, b_expert, w_out, b_out):
    grid = (_N // _BLK,)
    d_out = w_out.shape[1]
    return pl.pallas_call(
        _moe_body,
        grid=grid,
        in_specs=[
            pl.BlockSpec((2, _NCORE, _BLK, _HH), lambda i: (0, 0, i, 0)),
            pl.BlockSpec((_NCORE, _BLK, 16), lambda i: (0, i, 0)),
            pl.BlockSpec((_BLK, _NE), lambda i: (i, 0)),
            pl.BlockSpec((2, _NE), lambda i: (0, 0)),
            pl.BlockSpec((_NE, _HID, _HID), lambda i: (0, 0, 0)),
            pl.BlockSpec((_NE, _HID), lambda i: (0, 0)),
            pl.BlockSpec((_HID, d_out), lambda i: (0, 0)),
            pl.BlockSpec((1, d_out), lambda i: (0, 0)),
        ],
        out_specs=[
            pl.BlockSpec((_BLK, d_out), lambda i: (i, 0)),
            pl.BlockSpec((1, 1), lambda i: (0, 0)),
        ],
        out_shape=[
            jax.ShapeDtypeStruct((_N, d_out), jnp.float32),
            jax.ShapeDtypeStruct((1, 1), jnp.float32),
        ],
    )(aggp, degp, gates, implo, w_expert, b_expert, w_out, b_out)


def kernel(x, edge_index, noise, W_in, b_in, w_gate, w_noise,
           W_expert, b_expert, W_out, b_out):
    h2, raw_logits, gates, implo = _gating_call(
        x, noise, W_in, b_in.reshape(1, -1), w_gate, w_noise)
    ei = edge_index.reshape(2, _NW, _CH, _K)
    z64 = jnp.zeros((_RPT, _HH), jnp.float32)
    z16 = jnp.zeros((_RPT, 16), jnp.float32)
    ones16 = jnp.ones((_K, 16), jnp.float32)
    aggp, degp = _sc_aggregate(h2, ei, z64, z16, ones16)
    out, lb = _moe_call(aggp, degp, gates, implo,
                        W_expert, b_expert, W_out, b_out.reshape(1, -1))
    return out, lb.reshape(()), raw_logits


# speedup vs baseline: 1.9229x; 1.0404x over previous
import functools
import math

import jax
import jax.numpy as jnp
from jax import lax
from jax.experimental import pallas as pl
from jax.experimental.pallas import tpu as pltpu
from jax.experimental.pallas import tpu_sc as plsc

_N = 10000
_E = 320000
_HID = 128
_HH = _HID // 2
_NE = 8
_NCORE = 2
_NSUB = 16
_NW = _NCORE * _NSUB
_EPW = _E // _NW
_K = 80
_CH = _EPW // _K
_NP = 10240
_RPT = _NP // _NSUB
_BLK = 1000
_COEF = 0.01
_NUM_LAYERS = 4


def _h_body(x_ref, win_ref, bin_ref, h_ref):
    h = jnp.maximum(
        jnp.dot(x_ref[...], win_ref[...], preferred_element_type=jnp.float32)
        + bin_ref[...], 0.0)
    h_ref[0] = h[:, :_HH]
    h_ref[1] = h[:, _HH:]


def _h_call(x, w_in, b_in):
    grid = (_N // _BLK,)
    return pl.pallas_call(
        _h_body,
        grid=grid,
        in_specs=[
            pl.BlockSpec((_BLK, _HID), lambda i: (i, 0)),
            pl.BlockSpec((_HID, _HID), lambda i: (0, 0)),
            pl.BlockSpec((1, _HID), lambda i: (0, 0)),
        ],
        out_specs=[pl.BlockSpec((2, _BLK, _HH), lambda i: (0, i, 0))],
        out_shape=[jax.ShapeDtypeStruct((2, _N, _HH), jnp.float32)],
    )(x, w_in, b_in)[0]


def _gate_body(h2_ref, noise_ref, wg_ref, wn_ref,
               raw_ref, gates_ref, implo_ref):
    h = jnp.concatenate([h2_ref[0], h2_ref[1]], axis=1)
    clean = jnp.dot(h, wg_ref[...], preferred_element_type=jnp.float32)
    sp = jnp.dot(h, wn_ref[...], preferred_element_type=jnp.float32)
    nstd = jnp.maximum(sp, 0.0) + jnp.log1p(jnp.exp(-jnp.abs(sp))) + 1e-2
    raw = clean + noise_ref[...] * nstd
    raw_ref[...] = raw

    ids = lax.broadcasted_iota(jnp.int32, raw.shape, 1)
    v1 = jnp.max(raw, axis=1, keepdims=True)
    i1 = jnp.min(jnp.where(raw >= v1, ids, _NE), axis=1, keepdims=True)
    masked = jnp.where(ids == i1, -jnp.inf, raw)
    v2 = jnp.max(masked, axis=1, keepdims=True)
    i2 = jnp.min(jnp.where(masked >= v2, ids, _NE), axis=1, keepdims=True)
    e = jnp.exp(v2 - v1)
    g1 = 1.0 / (1.0 + e)
    g2 = e / (1.0 + e)
    gates = jnp.where(ids == i1, g1, 0.0) + jnp.where(ids == i2, g2, 0.0)
    gates_ref[...] = gates

    @pl.when(pl.program_id(0) == 0)
    def _():
        implo_ref[...] = jnp.zeros_like(implo_ref)

    imp = jnp.sum(gates, axis=0)[None, :]
    load = jnp.sum((gates > 0.0).astype(jnp.float32), axis=0)[None, :]
    implo_ref[...] += jnp.concatenate([imp, load], axis=0)


def _gating_call(h2, noise, w_gate, w_noise):
    grid = (_N // _BLK,)
    return pl.pallas_call(
        _gate_body,
        grid=grid,
        in_specs=[
            pl.BlockSpec((2, _BLK, _HH), lambda i: (0, i, 0)),
            pl.BlockSpec((_BLK, _NE), lambda i: (i, 0)),
            pl.BlockSpec((_HID, _NE), lambda i: (0, 0)),
            pl.BlockSpec((_HID, _NE), lambda i: (0, 0)),
        ],
        out_specs=[
            pl.BlockSpec((_BLK, _NE), lambda i: (i, 0)),
            pl.BlockSpec((_BLK, _NE), lambda i: (i, 0)),
            pl.BlockSpec((2, _NE), lambda i: (0, 0)),
        ],
        out_shape=[
            jax.ShapeDtypeStruct((_N, _NE), jnp.float32),
            jax.ShapeDtypeStruct((_N, _NE), jnp.float32),
            jax.ShapeDtypeStruct((2, _NE), jnp.float32),
        ],
    )(h2, noise, w_gate, w_noise)


def _sc_aggregate(h2, ei, z64, z16, ones16):
    mesh = plsc.VectorSubcoreMesh(core_axis_name="c", subcore_axis_name="s")

    @functools.partial(
        pl.kernel,
        out_type=[
            jax.ShapeDtypeStruct((2, _NCORE, _NP, _HH), jnp.float32),
            jax.ShapeDtypeStruct((_NCORE, _NP, 16), jnp.float32),
        ],
        mesh=mesh,
        compiler_params=pltpu.CompilerParams(use_tc_tiling_on_sc=False),
        scratch_types=[
            pltpu.VMEM((_CH, _K), jnp.int32),
            pltpu.VMEM((_CH, _K), jnp.int32),
            [pltpu.VMEM((_K, _HH), jnp.float32) for _ in range(2)],
            pltpu.VMEM((_K, 16), jnp.float32),
            pltpu.VMEM_SHARED((_NP, _HH), jnp.float32),
            pltpu.VMEM_SHARED((_NP, 16), jnp.float32),
            [pltpu.SemaphoreType.DMA for _ in range(2)],
            [pltpu.SemaphoreType.DMA for _ in range(2)],
        ],
    )
    def body(h_hbm, ei_hbm, z64_hbm, z16_hbm, ones_hbm,
             aggp_hbm, degp_hbm,
             sidx, didx, rows, ones_v, agg_sh, deg_sh, gsem, ssem):
        cid = lax.axis_index("c")
        sid = lax.axis_index("s")
        w = cid * _NSUB + sid
        rbase = sid * _RPT

        pltpu.sync_copy(ones_hbm, ones_v)
        pltpu.sync_copy(ei_hbm.at[0, w], sidx)
        pltpu.sync_copy(ei_hbm.at[1, w], didx)

        for half in range(2):
            first = half == 0
            htab = h_hbm.at[half]

            pltpu.sync_copy(z64_hbm, agg_sh.at[pl.ds(rbase, _RPT)])
            if first:
                pltpu.sync_copy(z16_hbm, deg_sh.at[pl.ds(rbase, _RPT)])
            plsc.subcore_barrier()

            def start_gather(c, rows, sem):
                pltpu.async_copy(htab.at[sidx.at[c]], rows, sem)

            def wait_gather(c, rows, sem):
                pltpu.make_async_copy(htab.at[sidx.at[c]], rows, sem).wait()

            def start_scatter(c, rows, sem):
                pltpu.async_copy(rows, agg_sh.at[didx.at[c]], sem, add=True)
                if first:
                    pltpu.async_copy(ones_v, deg_sh.at[didx.at[c]], sem,
                                     add=True)

            def wait_scatter(c, rows, sem):
                pltpu.make_async_copy(rows, agg_sh.at[didx.at[c]], sem).wait()
                if first:
                    pltpu.make_async_copy(ones_v, deg_sh.at[didx.at[c]],
                                          sem).wait()

            start_gather(0, rows[0], gsem[0])

            def step(i, carry):
                c0 = 2 * i
                c1 = c0 + 1
                c2 = c0 + 2
                wait_gather(c0, rows[0], gsem[0])
                start_scatter(c0, rows[0], ssem[0])
                start_gather(c1, rows[1], gsem[1])
                wait_gather(c1, rows[1], gsem[1])
                start_scatter(c1, rows[1], ssem[1])
                wait_scatter(c0, rows[0], ssem[0])
                start_gather(c2, rows[0], gsem[0])
                wait_scatter(c1, rows[1], ssem[1])
                return carry

            lax.fori_loop(0, (_CH - 1) // 2, step, 0)

            last = _CH - 1
            wait_gather(last, rows[0], gsem[0])
            start_scatter(last, rows[0], ssem[0])
            wait_scatter(last, rows[0], ssem[0])

            plsc.subcore_barrier()
            pltpu.sync_copy(agg_sh.at[pl.ds(rbase, _RPT)],
                            aggp_hbm.at[half, cid, pl.ds(rbase, _RPT)])
            if first:
                pltpu.sync_copy(deg_sh.at[pl.ds(rbase, _RPT)],
                                degp_hbm.at[cid, pl.ds(rbase, _RPT)])

    return body(h2, ei, z64, z16, ones16)


def _moe_body(p_ref, degp_ref, gates_ref, implo_ref, we_ref, be_ref,
              wo_ref, bo_ref, out_ref, lb_ref):
    p = p_ref[...]
    agg_sum = jnp.concatenate([p[0, 0] + p[0, 1], p[1, 0] + p[1, 1]], axis=1)
    deg = degp_ref[0, :, 0] + degp_ref[1, :, 0]
    agg = agg_sum / jnp.maximum(deg, 1.0)[:, None]
    gates = gates_ref[...]
    y = jnp.zeros_like(agg)
    for e in range(_NE):
        eo = jnp.maximum(
            jnp.dot(agg, we_ref[e], preferred_element_type=jnp.float32)
            + be_ref[e][None, :], 0.0)
        y = y + gates[:, e][:, None] * eo
    out_ref[...] = (jnp.dot(y, wo_ref[...], preferred_element_type=jnp.float32)
                    + bo_ref[...])

    @pl.when(pl.program_id(0) == 0)
    def _():
        implo = implo_ref[...]

        def cv2(v):
            m = jnp.mean(v)
            var = jnp.sum((v - m) ** 2) / (v.shape[0] - 1)
            return var / (m * m + 1e-10)

        scale = _COEF / math.ceil((_NUM_LAYERS - 2) / 2)
        lb_ref[...] = (scale * (cv2(implo[0]) + cv2(implo[1]))).reshape(1, 1)


def _moe_call(aggp, degp, gates, implo, w_expert, b_expert, w_out, b_out):
    grid = (_N // _BLK,)
    d_out = w_out.shape[1]
    return pl.pallas_call(
        _moe_body,
        grid=grid,
        in_specs=[
            pl.BlockSpec((2, _NCORE, _BLK, _HH), lambda i: (0, 0, i, 0)),
            pl.BlockSpec((_NCORE, _BLK, 16), lambda i: (0, i, 0)),
            pl.BlockSpec((_BLK, _NE), lambda i: (i, 0)),
            pl.BlockSpec((2, _NE), lambda i: (0, 0)),
            pl.BlockSpec((_NE, _HID, _HID), lambda i: (0, 0, 0)),
            pl.BlockSpec((_NE, _HID), lambda i: (0, 0)),
            pl.BlockSpec((_HID, d_out), lambda i: (0, 0)),
            pl.BlockSpec((1, d_out), lambda i: (0, 0)),
        ],
        out_specs=[
            pl.BlockSpec((_BLK, d_out), lambda i: (i, 0)),
            pl.BlockSpec((1, 1), lambda i: (0, 0)),
        ],
        out_shape=[
            jax.ShapeDtypeStruct((_N, d_out), jnp.float32),
            jax.ShapeDtypeStruct((1, 1), jnp.float32),
        ],
    )(aggp, degp, gates, implo, w_expert, b_expert, w_out, b_out)


def kernel(x, edge_index, noise, W_in, b_in, w_gate, w_noise,
           W_expert, b_expert, W_out, b_out):
    h2 = _h_call(x, W_in, b_in.reshape(1, -1))
    raw_logits, gates, implo = _gating_call(h2, noise, w_gate, w_noise)
    ei = edge_index.reshape(2, _NW, _CH, _K)
    z64 = jnp.zeros((_RPT, _HH), jnp.float32)
    z16 = jnp.zeros((_RPT, 16), jnp.float32)
    ones16 = jnp.ones((_K, 16), jnp.float32)
    aggp, degp = _sc_aggregate(h2, ei, z64, z16, ones16)
    out, lb = _moe_call(aggp, degp, gates, implo,
                        W_expert, b_expert, W_out, b_out.reshape(1, -1))
    return out, lb.reshape(()), raw_logits


# speedup vs baseline: 2.5808x; 1.3422x over previous
import functools
import math

import jax
import jax.numpy as jnp
from jax import lax
from jax.experimental import pallas as pl
from jax.experimental.pallas import tpu as pltpu
from jax.experimental.pallas import tpu_sc as plsc

_N = 10000
_E = 320000
_HID = 128
_HH = _HID // 2
_NE = 8
_NCORE = 2
_NSUB = 16
_NW = _NCORE * _NSUB
_EPW = _E // _NW
_K = 80
_CH = _EPW // _K
_NP = 10240
_RPT = _NP // _NSUB
_BLK = 1000
_COEF = 0.01
_NUM_LAYERS = 4


def _h_body(x_ref, win_ref, bin_ref, h_ref):
    h = jnp.maximum(
        jnp.dot(x_ref[...], win_ref[...], preferred_element_type=jnp.float32)
        + bin_ref[...], 0.0)
    h_ref[0] = h[:, :_HH]
    h_ref[1] = h[:, _HH:]


def _h_call(x, w_in, b_in):
    grid = (_N // _BLK,)
    return pl.pallas_call(
        _h_body,
        grid=grid,
        in_specs=[
            pl.BlockSpec((_BLK, _HID), lambda i: (i, 0)),
            pl.BlockSpec((_HID, _HID), lambda i: (0, 0)),
            pl.BlockSpec((1, _HID), lambda i: (0, 0)),
        ],
        out_specs=[pl.BlockSpec((2, _BLK, _HH), lambda i: (0, i, 0))],
        out_shape=[jax.ShapeDtypeStruct((2, _N, _HH), jnp.float32)],
    )(x, w_in, b_in)[0]


def _gate_body(h2_ref, noise_ref, wg_ref, wn_ref,
               raw_ref, gates_ref, implo_ref):
    h = jnp.concatenate([h2_ref[0], h2_ref[1]], axis=1)
    clean = jnp.dot(h, wg_ref[...], preferred_element_type=jnp.float32)
    sp = jnp.dot(h, wn_ref[...], preferred_element_type=jnp.float32)
    nstd = jnp.maximum(sp, 0.0) + jnp.log1p(jnp.exp(-jnp.abs(sp))) + 1e-2
    raw = clean + noise_ref[...] * nstd
    raw_ref[...] = raw

    ids = lax.broadcasted_iota(jnp.int32, raw.shape, 1)
    v1 = jnp.max(raw, axis=1, keepdims=True)
    i1 = jnp.min(jnp.where(raw >= v1, ids, _NE), axis=1, keepdims=True)
    masked = jnp.where(ids == i1, -jnp.inf, raw)
    v2 = jnp.max(masked, axis=1, keepdims=True)
    i2 = jnp.min(jnp.where(masked >= v2, ids, _NE), axis=1, keepdims=True)
    e = jnp.exp(v2 - v1)
    g1 = 1.0 / (1.0 + e)
    g2 = e / (1.0 + e)
    gates = jnp.where(ids == i1, g1, 0.0) + jnp.where(ids == i2, g2, 0.0)
    gates_ref[...] = gates

    @pl.when(pl.program_id(0) == 0)
    def _():
        implo_ref[...] = jnp.zeros_like(implo_ref)

    imp = jnp.sum(gates, axis=0)[None, :]
    load = jnp.sum((gates > 0.0).astype(jnp.float32), axis=0)[None, :]
    implo_ref[...] += jnp.concatenate([imp, load], axis=0)


def _gating_call(h2, noise, w_gate, w_noise):
    grid = (_N // _BLK,)
    return pl.pallas_call(
        _gate_body,
        grid=grid,
        in_specs=[
            pl.BlockSpec((2, _BLK, _HH), lambda i: (0, i, 0)),
            pl.BlockSpec((_BLK, _NE), lambda i: (i, 0)),
            pl.BlockSpec((_HID, _NE), lambda i: (0, 0)),
            pl.BlockSpec((_HID, _NE), lambda i: (0, 0)),
        ],
        out_specs=[
            pl.BlockSpec((_BLK, _NE), lambda i: (i, 0)),
            pl.BlockSpec((_BLK, _NE), lambda i: (i, 0)),
            pl.BlockSpec((2, _NE), lambda i: (0, 0)),
        ],
        out_shape=[
            jax.ShapeDtypeStruct((_N, _NE), jnp.float32),
            jax.ShapeDtypeStruct((_N, _NE), jnp.float32),
            jax.ShapeDtypeStruct((2, _NE), jnp.float32),
        ],
    )(h2, noise, w_gate, w_noise)


def _sc_aggregate(h2, ei, z64, z16, ones16):
    mesh = plsc.VectorSubcoreMesh(core_axis_name="c", subcore_axis_name="s")

    @functools.partial(
        pl.kernel,
        out_type=[
            jax.ShapeDtypeStruct((2, _NCORE, _NP, _HH), jnp.float32),
            jax.ShapeDtypeStruct((_NCORE, _NP, 16), jnp.float32),
        ],
        mesh=mesh,
        compiler_params=pltpu.CompilerParams(use_tc_tiling_on_sc=False),
        scratch_types=[
            pltpu.VMEM((_CH, _K), jnp.int32),
            pltpu.VMEM((_CH, _K), jnp.int32),
            [pltpu.VMEM((_K, _HH), jnp.float32) for _ in range(3)],
            pltpu.VMEM((_K, 16), jnp.float32),
            pltpu.VMEM_SHARED((_NP, _HH), jnp.float32),
            pltpu.VMEM_SHARED((_NP, 16), jnp.float32),
            [pltpu.SemaphoreType.DMA for _ in range(3)],
            [pltpu.SemaphoreType.DMA for _ in range(3)],
        ],
    )
    def body(h_hbm, ei_hbm, z64_hbm, z16_hbm, ones_hbm,
             aggp_hbm, degp_hbm,
             sidx, didx, rows, ones_v, agg_sh, deg_sh, gsem, ssem):
        cid = lax.axis_index("c")
        sid = lax.axis_index("s")
        w = cid * _NSUB + sid
        rbase = sid * _RPT

        pltpu.sync_copy(ones_hbm, ones_v)
        pltpu.sync_copy(ei_hbm.at[0, w], sidx)
        pltpu.sync_copy(ei_hbm.at[1, w], didx)

        for half in range(2):
            first = half == 0
            htab = h_hbm.at[half]

            pltpu.sync_copy(z64_hbm, agg_sh.at[pl.ds(rbase, _RPT)])
            if first:
                pltpu.sync_copy(z16_hbm, deg_sh.at[pl.ds(rbase, _RPT)])
            plsc.subcore_barrier()

            def start_gather(c, rows, sem):
                pltpu.async_copy(htab.at[sidx.at[c]], rows, sem)

            def wait_gather(c, rows, sem):
                pltpu.make_async_copy(htab.at[sidx.at[c]], rows, sem).wait()

            def start_scatter(c, rows, sem):
                pltpu.async_copy(rows, agg_sh.at[didx.at[c]], sem, add=True)
                if first:
                    pltpu.async_copy(ones_v, deg_sh.at[didx.at[c]], sem,
                                     add=True)

            def wait_scatter(c, rows, sem):
                pltpu.make_async_copy(rows, agg_sh.at[didx.at[c]], sem).wait()
                if first:
                    pltpu.make_async_copy(ones_v, deg_sh.at[didx.at[c]],
                                          sem).wait()

            for j in range(3):
                start_gather(j, rows[j], gsem[j])

            def step(i, carry):
                c = 3 * i
                wait_gather(c, rows[0], gsem[0])
                start_scatter(c, rows[0], ssem[0])
                wait_gather(c + 1, rows[1], gsem[1])
                start_scatter(c + 1, rows[1], ssem[1])
                wait_scatter(c, rows[0], ssem[0])
                start_gather(c + 3, rows[0], gsem[0])
                wait_gather(c + 2, rows[2], gsem[2])
                start_scatter(c + 2, rows[2], ssem[2])
                wait_scatter(c + 1, rows[1], ssem[1])
                start_gather(c + 4, rows[1], gsem[1])
                wait_scatter(c + 2, rows[2], ssem[2])
                start_gather(c + 5, rows[2], gsem[2])
                return carry

            lax.fori_loop(0, _CH // 3 - 1, step, 0)

            c = _CH - 5
            wait_gather(c, rows[0], gsem[0])
            start_scatter(c, rows[0], ssem[0])
            wait_gather(c + 1, rows[1], gsem[1])
            start_scatter(c + 1, rows[1], ssem[1])
            wait_scatter(c, rows[0], ssem[0])
            start_gather(c + 3, rows[0], gsem[0])
            wait_gather(c + 2, rows[2], gsem[2])
            start_scatter(c + 2, rows[2], ssem[2])
            wait_scatter(c + 1, rows[1], ssem[1])
            start_gather(c + 4, rows[1], gsem[1])
            wait_scatter(c + 2, rows[2], ssem[2])
            wait_gather(c + 3, rows[0], gsem[0])
            start_scatter(c + 3, rows[0], ssem[0])
            wait_gather(c + 4, rows[1], gsem[1])
            start_scatter(c + 4, rows[1], ssem[1])
            wait_scatter(c + 3, rows[0], ssem[0])
            wait_scatter(c + 4, rows[1], ssem[1])

            plsc.subcore_barrier()
            pltpu.sync_copy(agg_sh.at[pl.ds(rbase, _RPT)],
                            aggp_hbm.at[half, cid, pl.ds(rbase, _RPT)])
            if first:
                pltpu.sync_copy(deg_sh.at[pl.ds(rbase, _RPT)],
                                degp_hbm.at[cid, pl.ds(rbase, _RPT)])

    return body(h2, ei, z64, z16, ones16)


def _moe_body(p_ref, degp_ref, gates_ref, implo_ref, we_ref, be_ref,
              wo_ref, bo_ref, out_ref, lb_ref):
    p = p_ref[...]
    agg_sum = jnp.concatenate([p[0, 0] + p[0, 1], p[1, 0] + p[1, 1]], axis=1)
    deg = degp_ref[0, :, 0] + degp_ref[1, :, 0]
    agg = agg_sum / jnp.maximum(deg, 1.0)[:, None]
    gates = gates_ref[...]
    y = jnp.zeros_like(agg)
    for e in range(_NE):
        eo = jnp.maximum(
            jnp.dot(agg, we_ref[e], preferred_element_type=jnp.float32)
            + be_ref[e][None, :], 0.0)
        y = y + gates[:, e][:, None] * eo
    out_ref[...] = (jnp.dot(y, wo_ref[...], preferred_element_type=jnp.float32)
                    + bo_ref[...])

    @pl.when(pl.program_id(0) == 0)
    def _():
        implo = implo_ref[...]

        def cv2(v):
            m = jnp.mean(v)
            var = jnp.sum((v - m) ** 2) / (v.shape[0] - 1)
            return var / (m * m + 1e-10)

        scale = _COEF / math.ceil((_NUM_LAYERS - 2) / 2)
        lb_ref[...] = (scale * (cv2(implo[0]) + cv2(implo[1]))).reshape(1, 1)


def _moe_call(aggp, degp, gates, implo, w_expert, b_expert, w_out, b_out):
    grid = (_N // _BLK,)
    d_out = w_out.shape[1]
    return pl.pallas_call(
        _moe_body,
        grid=grid,
        in_specs=[
            pl.BlockSpec((2, _NCORE, _BLK, _HH), lambda i: (0, 0, i, 0)),
            pl.BlockSpec((_NCORE, _BLK, 16), lambda i: (0, i, 0)),
            pl.BlockSpec((_BLK, _NE), lambda i: (i, 0)),
            pl.BlockSpec((2, _NE), lambda i: (0, 0)),
            pl.BlockSpec((_NE, _HID, _HID), lambda i: (0, 0, 0)),
            pl.BlockSpec((_NE, _HID), lambda i: (0, 0)),
            pl.BlockSpec((_HID, d_out), lambda i: (0, 0)),
            pl.BlockSpec((1, d_out), lambda i: (0, 0)),
        ],
        out_specs=[
            pl.BlockSpec((_BLK, d_out), lambda i: (i, 0)),
            pl.BlockSpec((1, 1), lambda i: (0, 0)),
        ],
        out_shape=[
            jax.ShapeDtypeStruct((_N, d_out), jnp.float32),
            jax.ShapeDtypeStruct((1, 1), jnp.float32),
        ],
    )(aggp, degp, gates, implo, w_expert, b_expert, w_out, b_out)


def kernel(x, edge_index, noise, W_in, b_in, w_gate, w_noise,
           W_expert, b_expert, W_out, b_out):
    h2 = _h_call(x, W_in, b_in.reshape(1, -1))
    raw_logits, gates, implo = _gating_call(h2, noise, w_gate, w_noise)
    ei = edge_index.reshape(2, _NW, _CH, _K)
    z64 = jnp.zeros((_RPT, _HH), jnp.float32)
    z16 = jnp.zeros((_RPT, 16), jnp.float32)
    ones16 = jnp.ones((_K, 16), jnp.float32)
    aggp, degp = _sc_aggregate(h2, ei, z64, z16, ones16)
    out, lb = _moe_call(aggp, degp, gates, implo,
                        W_expert, b_expert, W_out, b_out.reshape(1, -1))
    return out, lb.reshape(()), raw_logits


# speedup vs baseline: 2.9998x; 1.1623x over previous
import functools
import math

import jax
import jax.numpy as jnp
from jax import lax
from jax.experimental import pallas as pl
from jax.experimental.pallas import tpu as pltpu
from jax.experimental.pallas import tpu_sc as plsc

_N = 10000
_E = 320000
_HID = 128
_HH = _HID // 2
_NE = 8
_NCORE = 2
_NSUB = 16
_NW = _NCORE * _NSUB
_EPW = _E // _NW
_K = 80
_CH = _EPW // _K
_NP = 10240
_RPT = _NP // _NSUB
_BLK = 1000
_COEF = 0.01
_NUM_LAYERS = 4


def _h_body(x_ref, win_ref, bin_ref, h_ref):
    h = jnp.maximum(
        jnp.dot(x_ref[...], win_ref[...], preferred_element_type=jnp.float32)
        + bin_ref[...], 0.0)
    h_ref[0] = h[:, :_HH]
    h_ref[1] = h[:, _HH:]


def _h_call(x, w_in, b_in):
    grid = (_N // _BLK,)
    return pl.pallas_call(
        _h_body,
        grid=grid,
        in_specs=[
            pl.BlockSpec((_BLK, _HID), lambda i: (i, 0)),
            pl.BlockSpec((_HID, _HID), lambda i: (0, 0)),
            pl.BlockSpec((1, _HID), lambda i: (0, 0)),
        ],
        out_specs=[pl.BlockSpec((2, _BLK, _HH), lambda i: (0, i, 0))],
        out_shape=[jax.ShapeDtypeStruct((2, _N, _HH), jnp.float32)],
    )(x, w_in, b_in)[0]


def _gate_body(h2_ref, noise_ref, wg_ref, wn_ref,
               raw_ref, gates_ref, implo_ref):
    h = jnp.concatenate([h2_ref[0], h2_ref[1]], axis=1)
    clean = jnp.dot(h, wg_ref[...], preferred_element_type=jnp.float32)
    sp = jnp.dot(h, wn_ref[...], preferred_element_type=jnp.float32)
    nstd = jnp.maximum(sp, 0.0) + jnp.log1p(jnp.exp(-jnp.abs(sp))) + 1e-2
    raw = clean + noise_ref[...] * nstd
    raw_ref[...] = raw

    ids = lax.broadcasted_iota(jnp.int32, raw.shape, 1)
    v1 = jnp.max(raw, axis=1, keepdims=True)
    i1 = jnp.min(jnp.where(raw >= v1, ids, _NE), axis=1, keepdims=True)
    masked = jnp.where(ids == i1, -jnp.inf, raw)
    v2 = jnp.max(masked, axis=1, keepdims=True)
    i2 = jnp.min(jnp.where(masked >= v2, ids, _NE), axis=1, keepdims=True)
    e = jnp.exp(v2 - v1)
    g1 = 1.0 / (1.0 + e)
    g2 = e / (1.0 + e)
    gates = jnp.where(ids == i1, g1, 0.0) + jnp.where(ids == i2, g2, 0.0)
    gates_ref[...] = gates

    @pl.when(pl.program_id(0) == 0)
    def _():
        implo_ref[...] = jnp.zeros_like(implo_ref)

    imp = jnp.sum(gates, axis=0)[None, :]
    load = jnp.sum((gates > 0.0).astype(jnp.float32), axis=0)[None, :]
    implo_ref[...] += jnp.concatenate([imp, load], axis=0)


def _gating_call(h2, noise, w_gate, w_noise):
    grid = (_N // _BLK,)
    return pl.pallas_call(
        _gate_body,
        grid=grid,
        in_specs=[
            pl.BlockSpec((2, _BLK, _HH), lambda i: (0, i, 0)),
            pl.BlockSpec((_BLK, _NE), lambda i: (i, 0)),
            pl.BlockSpec((_HID, _NE), lambda i: (0, 0)),
            pl.BlockSpec((_HID, _NE), lambda i: (0, 0)),
        ],
        out_specs=[
            pl.BlockSpec((_BLK, _NE), lambda i: (i, 0)),
            pl.BlockSpec((_BLK, _NE), lambda i: (i, 0)),
            pl.BlockSpec((2, _NE), lambda i: (0, 0)),
        ],
        out_shape=[
            jax.ShapeDtypeStruct((_N, _NE), jnp.float32),
            jax.ShapeDtypeStruct((_N, _NE), jnp.float32),
            jax.ShapeDtypeStruct((2, _NE), jnp.float32),
        ],
    )(h2, noise, w_gate, w_noise)


def _sc_aggregate(h2, ei, z64, z16, ones16):
    mesh = plsc.VectorSubcoreMesh(core_axis_name="c", subcore_axis_name="s")

    @functools.partial(
        pl.kernel,
        out_type=[
            jax.ShapeDtypeStruct((2, _NCORE, _NP, _HH), jnp.float32),
            jax.ShapeDtypeStruct((_NCORE, _NP, 16), jnp.float32),
        ],
        mesh=mesh,
        compiler_params=pltpu.CompilerParams(use_tc_tiling_on_sc=False),
        scratch_types=[
            pltpu.VMEM((_CH, _K), jnp.int32),
            pltpu.VMEM((_CH, _K), jnp.int32),
            [pltpu.VMEM((_K, _HH), jnp.float32) for _ in range(5)],
            pltpu.VMEM((_K, 16), jnp.float32),
            pltpu.VMEM_SHARED((_NP, _HH), jnp.float32),
            pltpu.VMEM_SHARED((_NP, 16), jnp.float32),
            [pltpu.SemaphoreType.DMA for _ in range(5)],
            [pltpu.SemaphoreType.DMA for _ in range(5)],
        ],
    )
    def body(h_hbm, ei_hbm, z64_hbm, z16_hbm, ones_hbm,
             aggp_hbm, degp_hbm,
             sidx, didx, rows, ones_v, agg_sh, deg_sh, gsem, ssem):
        cid = lax.axis_index("c")
        sid = lax.axis_index("s")
        w = cid * _NSUB + sid
        rbase = sid * _RPT

        pltpu.sync_copy(ones_hbm, ones_v)
        pltpu.sync_copy(ei_hbm.at[0, w], sidx)
        pltpu.sync_copy(ei_hbm.at[1, w], didx)

        for half in range(2):
            first = half == 0
            htab = h_hbm.at[half]

            pltpu.sync_copy(z64_hbm, agg_sh.at[pl.ds(rbase, _RPT)])
            if first:
                pltpu.sync_copy(z16_hbm, deg_sh.at[pl.ds(rbase, _RPT)])
            plsc.subcore_barrier()

            def start_gather(c, rows, sem):
                pltpu.async_copy(htab.at[sidx.at[c]], rows, sem)

            def wait_gather(c, rows, sem):
                pltpu.make_async_copy(htab.at[sidx.at[c]], rows, sem).wait()

            def start_scatter(c, rows, sem):
                pltpu.async_copy(rows, agg_sh.at[didx.at[c]], sem, add=True)
                if first:
                    pltpu.async_copy(ones_v, deg_sh.at[didx.at[c]], sem,
                                     add=True)

            def wait_scatter(c, rows, sem):
                pltpu.make_async_copy(rows, agg_sh.at[didx.at[c]], sem).wait()
                if first:
                    pltpu.make_async_copy(ones_v, deg_sh.at[didx.at[c]],
                                          sem).wait()

            for j in range(5):
                start_gather(j, rows[j], gsem[j])

            def step(i, carry):
                c = 5 * i
                wait_gather(c, rows[0], gsem[0])
                start_scatter(c, rows[0], ssem[0])
                wait_gather(c + 1, rows[1], gsem[1])
                start_scatter(c + 1, rows[1], ssem[1])
                wait_scatter(c, rows[0], ssem[0])
                start_gather(c + 5, rows[0], gsem[0])
                wait_gather(c + 2, rows[2], gsem[2])
                start_scatter(c + 2, rows[2], ssem[2])
                wait_scatter(c + 1, rows[1], ssem[1])
                start_gather(c + 6, rows[1], gsem[1])
                wait_gather(c + 3, rows[3], gsem[3])
                start_scatter(c + 3, rows[3], ssem[3])
                wait_scatter(c + 2, rows[2], ssem[2])
                start_gather(c + 7, rows[2], gsem[2])
                wait_gather(c + 4, rows[4], gsem[4])
                start_scatter(c + 4, rows[4], ssem[4])
                wait_scatter(c + 3, rows[3], ssem[3])
                start_gather(c + 8, rows[3], gsem[3])
                wait_scatter(c + 4, rows[4], ssem[4])
                start_gather(c + 9, rows[4], gsem[4])
                return carry

            lax.fori_loop(0, _CH // 5 - 1, step, 0)

            c = _CH - 5
            wait_gather(c, rows[0], gsem[0])
            start_scatter(c, rows[0], ssem[0])
            wait_gather(c + 1, rows[1], gsem[1])
            start_scatter(c + 1, rows[1], ssem[1])
            wait_gather(c + 2, rows[2], gsem[2])
            start_scatter(c + 2, rows[2], ssem[2])
            wait_scatter(c, rows[0], ssem[0])
            wait_gather(c + 3, rows[3], gsem[3])
            start_scatter(c + 3, rows[3], ssem[3])
            wait_scatter(c + 1, rows[1], ssem[1])
            wait_gather(c + 4, rows[4], gsem[4])
            start_scatter(c + 4, rows[4], ssem[4])
            wait_scatter(c + 2, rows[2], ssem[2])
            wait_scatter(c + 3, rows[3], ssem[3])
            wait_scatter(c + 4, rows[4], ssem[4])

            plsc.subcore_barrier()
            pltpu.sync_copy(agg_sh.at[pl.ds(rbase, _RPT)],
                            aggp_hbm.at[half, cid, pl.ds(rbase, _RPT)])
            if first:
                pltpu.sync_copy(deg_sh.at[pl.ds(rbase, _RPT)],
                                degp_hbm.at[cid, pl.ds(rbase, _RPT)])

    return body(h2, ei, z64, z16, ones16)


def _moe_body(p_ref, degp_ref, gates_ref, implo_ref, we_ref, be_ref,
              wo_ref, bo_ref, out_ref, lb_ref):
    p = p_ref[...]
    agg_sum = jnp.concatenate([p[0, 0] + p[0, 1], p[1, 0] + p[1, 1]], axis=1)
    deg = degp_ref[0, :, 0] + degp_ref[1, :, 0]
    agg = agg_sum / jnp.maximum(deg, 1.0)[:, None]
    gates = gates_ref[...]
    y = jnp.zeros_like(agg)
    for e in range(_NE):
        eo = jnp.maximum(
            jnp.dot(agg, we_ref[e], preferred_element_type=jnp.float32)
            + be_ref[e][None, :], 0.0)
        y = y + gates[:, e][:, None] * eo
    out_ref[...] = (jnp.dot(y, wo_ref[...], preferred_element_type=jnp.float32)
                    + bo_ref[...])

    @pl.when(pl.program_id(0) == 0)
    def _():
        implo = implo_ref[...]

        def cv2(v):
            m = jnp.mean(v)
            var = jnp.sum((v - m) ** 2) / (v.shape[0] - 1)
            return var / (m * m + 1e-10)

        scale = _COEF / math.ceil((_NUM_LAYERS - 2) / 2)
        lb_ref[...] = (scale * (cv2(implo[0]) + cv2(implo[1]))).reshape(1, 1)


def _moe_call(aggp, degp, gates, implo, w_expert, b_expert, w_out, b_out):
    grid = (_N // _BLK,)
    d_out = w_out.shape[1]
    return pl.pallas_call(
        _moe_body,
        grid=grid,
        in_specs=[
            pl.BlockSpec((2, _NCORE, _BLK, _HH), lambda i: (0, 0, i, 0)),
            pl.BlockSpec((_NCORE, _BLK, 16), lambda i: (0, i, 0)),
            pl.BlockSpec((_BLK, _NE), lambda i: (i, 0)),
            pl.BlockSpec((2, _NE), lambda i: (0, 0)),
            pl.BlockSpec((_NE, _HID, _HID), lambda i: (0, 0, 0)),
            pl.BlockSpec((_NE, _HID), lambda i: (0, 0)),
            pl.BlockSpec((_HID, d_out), lambda i: (0, 0)),
            pl.BlockSpec((1, d_out), lambda i: (0, 0)),
        ],
        out_specs=[
            pl.BlockSpec((_BLK, d_out), lambda i: (i, 0)),
            pl.BlockSpec((1, 1), lambda i: (0, 0)),
        ],
        out_shape=[
            jax.ShapeDtypeStruct((_N, d_out), jnp.float32),
            jax.ShapeDtypeStruct((1, 1), jnp.float32),
        ],
    )(aggp, degp, gates, implo, w_expert, b_expert, w_out, b_out)


def kernel(x, edge_index, noise, W_in, b_in, w_gate, w_noise,
           W_expert, b_expert, W_out, b_out):
    h2 = _h_call(x, W_in, b_in.reshape(1, -1))
    raw_logits, gates, implo = _gating_call(h2, noise, w_gate, w_noise)
    ei = edge_index.reshape(2, _NW, _CH, _K)
    z64 = jnp.zeros((_RPT, _HH), jnp.float32)
    z16 = jnp.zeros((_RPT, 16), jnp.float32)
    ones16 = jnp.ones((_K, 16), jnp.float32)
    aggp, degp = _sc_aggregate(h2, ei, z64, z16, ones16)
    out, lb = _moe_call(aggp, degp, gates, implo,
                        W_expert, b_expert, W_out, b_out.reshape(1, -1))
    return out, lb.reshape(()), raw_logits


# speedup vs baseline: 3.0503x; 1.0168x over previous
import functools
import math

import jax
import jax.numpy as jnp
from jax import lax
from jax.experimental import pallas as pl
from jax.experimental.pallas import tpu as pltpu
from jax.experimental.pallas import tpu_sc as plsc

_N = 10000
_E = 320000
_HID = 128
_HH = _HID // 2
_NE = 8
_NCORE = 2
_NSUB = 16
_NW = _NCORE * _NSUB
_EPW = _E // _NW
_K = 80
_CH = _EPW // _K
_NP = 10240
_RPT = _NP // _NSUB
_D = 8
_BLK = 1000
_COEF = 0.01
_NUM_LAYERS = 4


def _h_body(x_ref, win_ref, bin_ref, h_ref):
    h = jnp.maximum(
        jnp.dot(x_ref[...], win_ref[...], preferred_element_type=jnp.float32)
        + bin_ref[...], 0.0)
    h_ref[0] = h[:, :_HH]
    h_ref[1] = h[:, _HH:]


def _h_call(x, w_in, b_in):
    grid = (_N // _BLK,)
    return pl.pallas_call(
        _h_body,
        grid=grid,
        in_specs=[
            pl.BlockSpec((_BLK, _HID), lambda i: (i, 0)),
            pl.BlockSpec((_HID, _HID), lambda i: (0, 0)),
            pl.BlockSpec((1, _HID), lambda i: (0, 0)),
        ],
        out_specs=[pl.BlockSpec((2, _BLK, _HH), lambda i: (0, i, 0))],
        out_shape=[jax.ShapeDtypeStruct((2, _N, _HH), jnp.float32)],
    )(x, w_in, b_in)[0]


def _gate_body(h2_ref, noise_ref, wg_ref, wn_ref,
               raw_ref, gates_ref, implo_ref):
    h = jnp.concatenate([h2_ref[0], h2_ref[1]], axis=1)
    clean = jnp.dot(h, wg_ref[...], preferred_element_type=jnp.float32)
    sp = jnp.dot(h, wn_ref[...], preferred_element_type=jnp.float32)
    nstd = jnp.maximum(sp, 0.0) + jnp.log1p(jnp.exp(-jnp.abs(sp))) + 1e-2
    raw = clean + noise_ref[...] * nstd
    raw_ref[...] = raw

    ids = lax.broadcasted_iota(jnp.int32, raw.shape, 1)
    v1 = jnp.max(raw, axis=1, keepdims=True)
    i1 = jnp.min(jnp.where(raw >= v1, ids, _NE), axis=1, keepdims=True)
    masked = jnp.where(ids == i1, -jnp.inf, raw)
    v2 = jnp.max(masked, axis=1, keepdims=True)
    i2 = jnp.min(jnp.where(masked >= v2, ids, _NE), axis=1, keepdims=True)
    e = jnp.exp(v2 - v1)
    g1 = 1.0 / (1.0 + e)
    g2 = e / (1.0 + e)
    gates = jnp.where(ids == i1, g1, 0.0) + jnp.where(ids == i2, g2, 0.0)
    gates_ref[...] = gates

    @pl.when(pl.program_id(0) == 0)
    def _():
        implo_ref[...] = jnp.zeros_like(implo_ref)

    imp = jnp.sum(gates, axis=0)[None, :]
    load = jnp.sum((gates > 0.0).astype(jnp.float32), axis=0)[None, :]
    implo_ref[...] += jnp.concatenate([imp, load], axis=0)


def _gating_call(h2, noise, w_gate, w_noise):
    grid = (_N // _BLK,)
    return pl.pallas_call(
        _gate_body,
        grid=grid,
        in_specs=[
            pl.BlockSpec((2, _BLK, _HH), lambda i: (0, i, 0)),
            pl.BlockSpec((_BLK, _NE), lambda i: (i, 0)),
            pl.BlockSpec((_HID, _NE), lambda i: (0, 0)),
            pl.BlockSpec((_HID, _NE), lambda i: (0, 0)),
        ],
        out_specs=[
            pl.BlockSpec((_BLK, _NE), lambda i: (i, 0)),
            pl.BlockSpec((_BLK, _NE), lambda i: (i, 0)),
            pl.BlockSpec((2, _NE), lambda i: (0, 0)),
        ],
        out_shape=[
            jax.ShapeDtypeStruct((_N, _NE), jnp.float32),
            jax.ShapeDtypeStruct((_N, _NE), jnp.float32),
            jax.ShapeDtypeStruct((2, _NE), jnp.float32),
        ],
    )(h2, noise, w_gate, w_noise)


def _sc_aggregate(h2, ei, z64, z16, ones16):
    mesh = plsc.VectorSubcoreMesh(core_axis_name="c", subcore_axis_name="s")

    @functools.partial(
        pl.kernel,
        out_type=[
            jax.ShapeDtypeStruct((2, _NCORE, _NP, _HH), jnp.float32),
            jax.ShapeDtypeStruct((_NCORE, _NP, 16), jnp.float32),
        ],
        mesh=mesh,
        compiler_params=pltpu.CompilerParams(use_tc_tiling_on_sc=False),
        scratch_types=[
            pltpu.VMEM((_CH, _K), jnp.int32),
            pltpu.VMEM((_CH, _K), jnp.int32),
            [pltpu.VMEM((_K, _HH), jnp.float32) for _ in range(_D)],
            pltpu.VMEM((_K, 16), jnp.float32),
            pltpu.VMEM_SHARED((_NP, _HH), jnp.float32),
            pltpu.VMEM_SHARED((_NP, 16), jnp.float32),
            [pltpu.SemaphoreType.DMA for _ in range(_D)],
            [pltpu.SemaphoreType.DMA for _ in range(_D)],
        ],
    )
    def body(h_hbm, ei_hbm, z64_hbm, z16_hbm, ones_hbm,
             aggp_hbm, degp_hbm,
             sidx, didx, rows, ones_v, agg_sh, deg_sh, gsem, ssem):
        cid = lax.axis_index("c")
        sid = lax.axis_index("s")
        w = cid * _NSUB + sid
        rbase = sid * _RPT

        pltpu.sync_copy(ones_hbm, ones_v)
        pltpu.sync_copy(ei_hbm.at[0, w], sidx)
        pltpu.sync_copy(ei_hbm.at[1, w], didx)

        for half in range(2):
            first = half == 0
            htab = h_hbm.at[half]

            pltpu.sync_copy(z64_hbm, agg_sh.at[pl.ds(rbase, _RPT)])
            if first:
                pltpu.sync_copy(z16_hbm, deg_sh.at[pl.ds(rbase, _RPT)])
            plsc.subcore_barrier()

            def start_gather(c, rows, sem):
                pltpu.async_copy(htab.at[sidx.at[c]], rows, sem)

            def wait_gather(c, rows, sem):
                pltpu.make_async_copy(htab.at[sidx.at[c]], rows, sem).wait()

            def start_scatter(c, rows, sem):
                pltpu.async_copy(rows, agg_sh.at[didx.at[c]], sem, add=True)
                if first:
                    pltpu.async_copy(ones_v, deg_sh.at[didx.at[c]], sem,
                                     add=True)

            def wait_scatter(c, rows, sem):
                pltpu.make_async_copy(rows, agg_sh.at[didx.at[c]], sem).wait()
                if first:
                    pltpu.make_async_copy(ones_v, deg_sh.at[didx.at[c]],
                                          sem).wait()

            for j in range(_D):
                start_gather(j, rows[j], gsem[j])

            def step(i, carry):
                c = _D * i
                for k in range(_D):
                    wait_gather(c + k, rows[k], gsem[k])
                    start_scatter(c + k, rows[k], ssem[k])
                    if k >= 1:
                        wait_scatter(c + k - 1, rows[k - 1], ssem[k - 1])
                        start_gather(c + k - 1 + _D, rows[k - 1], gsem[k - 1])
                wait_scatter(c + _D - 1, rows[_D - 1], ssem[_D - 1])
                start_gather(c + 2 * _D - 1, rows[_D - 1], gsem[_D - 1])
                return carry

            nloop = (_CH - 2 * _D) // _D + 1
            lax.fori_loop(0, nloop, step, 0)

            t0 = nloop * _D
            for idx, t in enumerate(range(t0, _CH)):
                wait_gather(t, rows[t % _D], gsem[t % _D])
                start_scatter(t, rows[t % _D], ssem[t % _D])
                if idx >= 1:
                    wait_scatter(t - 1, rows[(t - 1) % _D], ssem[(t - 1) % _D])
                    gq = t - 1 + _D
                    if gq < _CH:
                        start_gather(gq, rows[gq % _D], gsem[gq % _D])
            wait_scatter(_CH - 1, rows[(_CH - 1) % _D], ssem[(_CH - 1) % _D])

            plsc.subcore_barrier()
            pltpu.sync_copy(agg_sh.at[pl.ds(rbase, _RPT)],
                            aggp_hbm.at[half, cid, pl.ds(rbase, _RPT)])
            if first:
                pltpu.sync_copy(deg_sh.at[pl.ds(rbase, _RPT)],
                                degp_hbm.at[cid, pl.ds(rbase, _RPT)])

    return body(h2, ei, z64, z16, ones16)


def _moe_body(p_ref, degp_ref, gates_ref, implo_ref, we_ref, be_ref,
              wo_ref, bo_ref, out_ref, lb_ref):
    p = p_ref[...]
    agg_sum = jnp.concatenate([p[0, 0] + p[0, 1], p[1, 0] + p[1, 1]], axis=1)
    deg = degp_ref[0, :, 0] + degp_ref[1, :, 0]
    agg = agg_sum / jnp.maximum(deg, 1.0)[:, None]
    gates = gates_ref[...]
    y = jnp.zeros_like(agg)
    for e in range(_NE):
        eo = jnp.maximum(
            jnp.dot(agg, we_ref[e], preferred_element_type=jnp.float32)
            + be_ref[e][None, :], 0.0)
        y = y + gates[:, e][:, None] * eo
    out_ref[...] = (jnp.dot(y, wo_ref[...], preferred_element_type=jnp.float32)
                    + bo_ref[...])

    @pl.when(pl.program_id(0) == 0)
    def _():
        implo = implo_ref[...]

        def cv2(v):
            m = jnp.mean(v)
            var = jnp.sum((v - m) ** 2) / (v.shape[0] - 1)
            return var / (m * m + 1e-10)

        scale = _COEF / math.ceil((_NUM_LAYERS - 2) / 2)
        lb_ref[...] = (scale * (cv2(implo[0]) + cv2(implo[1]))).reshape(1, 1)


def _moe_call(aggp, degp, gates, implo, w_expert, b_expert, w_out, b_out):
    grid = (_N // _BLK,)
    d_out = w_out.shape[1]
    return pl.pallas_call(
        _moe_body,
        grid=grid,
        in_specs=[
            pl.BlockSpec((2, _NCORE, _BLK, _HH), lambda i: (0, 0, i, 0)),
            pl.BlockSpec((_NCORE, _BLK, 16), lambda i: (0, i, 0)),
            pl.BlockSpec((_BLK, _NE), lambda i: (i, 0)),
            pl.BlockSpec((2, _NE), lambda i: (0, 0)),
            pl.BlockSpec((_NE, _HID, _HID), lambda i: (0, 0, 0)),
            pl.BlockSpec((_NE, _HID), lambda i: (0, 0)),
            pl.BlockSpec((_HID, d_out), lambda i: (0, 0)),
            pl.BlockSpec((1, d_out), lambda i: (0, 0)),
        ],
        out_specs=[
            pl.BlockSpec((_BLK, d_out), lambda i: (i, 0)),
            pl.BlockSpec((1, 1), lambda i: (0, 0)),
        ],
        out_shape=[
            jax.ShapeDtypeStruct((_N, d_out), jnp.float32),
            jax.ShapeDtypeStruct((1, 1), jnp.float32),
        ],
    )(aggp, degp, gates, implo, w_expert, b_expert, w_out, b_out)


def kernel(x, edge_index, noise, W_in, b_in, w_gate, w_noise,
           W_expert, b_expert, W_out, b_out):
    h2 = _h_call(x, W_in, b_in.reshape(1, -1))
    raw_logits, gates, implo = _gating_call(h2, noise, w_gate, w_noise)
    ei = edge_index.reshape(2, _NW, _CH, _K)
    z64 = jnp.zeros((_RPT, _HH), jnp.float32)
    z16 = jnp.zeros((_RPT, 16), jnp.float32)
    ones16 = jnp.ones((_K, 16), jnp.float32)
    aggp, degp = _sc_aggregate(h2, ei, z64, z16, ones16)
    out, lb = _moe_call(aggp, degp, gates, implo,
                        W_expert, b_expert, W_out, b_out.reshape(1, -1))
    return out, lb.reshape(()), raw_logits
